# baseline (speedup 1.0000x reference)
import jax
import jax.numpy as jnp
from jax.experimental import pallas as pl
from jax.experimental.pallas import tpu as pltpu

_B = 16384
_S = 7
_C = 30
_CELLS = _S * _S
_GX = 7.0
_GY = 30.0

_BL = 8192
_NL = _B // _BL
_CHUNK = 2048
_NCH = _BL // _CHUNK


def _iou_pair(pc0, pc1, pw, ph, lc0, lc1, lw, lh, a2):
    iw = jnp.minimum(jnp.minimum(
        0.5 * (pw + lw) - jnp.abs(pc0 - lc0) * (1.0 / _GX), pw), lw)
    ih = jnp.minimum(jnp.minimum(
        0.5 * (ph + lh) - jnp.abs(pc1 - lc1) * (1.0 / _GY), ph), lh)
    inter = jnp.maximum(iw, 0.0) * jnp.maximum(ih, 0.0)
    union = pw * ph + a2 - inter
    pos = inter > 0
    return jnp.where(pos, inter / jnp.where(pos, union, 1.0), 0.0)


def _yolo_kernel(p_ref, l_ref, out_ref):
    @pl.when(pl.program_id(1) == 0)
    def _():
        out_ref[...] = jnp.zeros_like(out_ref)

    def chunk_body(k):
        sl = pl.ds(k * _CHUNK, _CHUNK)

        def pch(c):
            return p_ref[:, c, sl]

        def lch(c):
            return l_ref[:, c, sl]

        dd = p_ref[:, 10:30, sl] - l_ref[:, 10:30, sl]
        cls = jnp.sum(dd * dd, axis=1)

        l0, l1, l2, l3 = lch(0), lch(1), lch(2), lch(3)
        a2 = l2 * l3

        p0, p1, p2, p3 = pch(0), pch(1), pch(2), pch(3)
        coor1 = ((p0 - l0) ** 2 + (p1 - l1) ** 2
                 + (jnp.sqrt(p2) - jnp.sqrt(l2)) ** 2
                 + (jnp.sqrt(p3) - jnp.sqrt(l3)) ** 2)
        iou1 = _iou_pair(p0, p1, p2, p3, l0, l1, l2, l3, a2)

        p5, p6, p7, p8 = pch(5), pch(6), pch(7), pch(8)
        coor2 = ((p5 - lch(5)) ** 2 + (p6 - lch(6)) ** 2
                 + (jnp.sqrt(p7) - jnp.sqrt(lch(7))) ** 2
                 + (jnp.sqrt(p8) - jnp.sqrt(lch(8))) ** 2)
        iou2 = _iou_pair(p5, p6, p7, p8, l0, l1, l2, l3, a2)

        sel1 = iou1 >= iou2

        p4, p9, l4 = pch(4), pch(9), lch(4)
        obj = l4 == 1.0
        d1 = (p4 - iou1) ** 2
        d2 = (p9 - iou2) ** 2
        obj_branch = (5.0 * jnp.where(sel1, coor1, coor2)
                      + jnp.where(sel1, d1, d2)
                      + 0.5 * jnp.where(sel1, d2, d1)
                      + cls)
        noobj = 0.5 * (p4 * p4 + p9 * p9)
        per_cell = jnp.where(obj, obj_branch, noobj)

        out_ref[0:1, sl] += jnp.sum(per_cell, axis=0, keepdims=True)

    def chunk_pair(k, carry):
        chunk_body(2 * k)
        chunk_body(2 * k + 1)
        return carry

    jax.lax.fori_loop(0, _NCH // 2, chunk_pair, 0)


def kernel(pred, labels):
    pt = jnp.transpose(pred, (1, 2, 3, 0)).reshape(_CELLS, _C, _B)
    lt = jnp.transpose(labels, (1, 2, 3, 0)).reshape(_CELLS, _C, _B)

    in_spec = pl.BlockSpec((_S, _C, _BL), lambda j, i: (i, 0, j))

    out = pl.pallas_call(
        _yolo_kernel,
        out_shape=jax.ShapeDtypeStruct((1, _B), jnp.float32),
        grid=(_NL, _S),
        in_specs=[in_spec, in_spec],
        out_specs=pl.BlockSpec((1, _BL), lambda j, i: (0, j)),
        compiler_params=pltpu.CompilerParams(
            dimension_semantics=("arbitrary", "arbitrary"),
            vmem_limit_bytes=56 * 1024 * 1024,
        ),
        name="yolo_v1_loss",
    )(pt, lt)

    return jnp.sum(out) * (1.0 / _B)


# speedup vs baseline: 1.0061x; 1.0061x over previous
import jax
import jax.numpy as jnp
from jax.experimental import pallas as pl
from jax.experimental.pallas import tpu as pltpu

_B = 16384
_S = 7
_C = 30
_CELLS = _S * _S
_GX = 7.0
_GY = 30.0

_BL = 8192
_NL = _B // _BL
_CHUNK = 2048
_NCH = _BL // _CHUNK


def _iou_pair(pc0, pc1, pw, ph, lc0, lc1, lw, lh, a2):
    iw = jnp.minimum(jnp.minimum(
        0.5 * (pw + lw) - jnp.abs(pc0 - lc0) * (1.0 / _GX), pw), lw)
    ih = jnp.minimum(jnp.minimum(
        0.5 * (ph + lh) - jnp.abs(pc1 - lc1) * (1.0 / _GY), ph), lh)
    inter = jnp.maximum(iw, 0.0) * jnp.maximum(ih, 0.0)
    union = pw * ph + a2 - inter
    pos = inter > 0
    return jnp.where(pos, inter / jnp.where(pos, union, 1.0), 0.0)


def _yolo_kernel(p_ref, l_ref, out_ref):
    @pl.when(pl.program_id(1) == 0)
    def _():
        out_ref[...] = jnp.zeros_like(out_ref)

    def chunk_body(k):
        sl = pl.ds(k * _CHUNK, _CHUNK)

        def pch(c):
            return p_ref[:, c, sl]

        def lch(c):
            return l_ref[:, c, sl]

        dd = p_ref[:, 10:30, sl] - l_ref[:, 10:30, sl]
        cls = jnp.sum(dd * dd, axis=1)

        l0, l1, l2, l3 = lch(0), lch(1), lch(2), lch(3)
        a2 = l2 * l3

        p0, p1, p2, p3 = pch(0), pch(1), pch(2), pch(3)
        coor1 = ((p0 - l0) ** 2 + (p1 - l1) ** 2
                 + (jnp.sqrt(p2) - jnp.sqrt(l2)) ** 2
                 + (jnp.sqrt(p3) - jnp.sqrt(l3)) ** 2)
        iou1 = _iou_pair(p0, p1, p2, p3, l0, l1, l2, l3, a2)

        p5, p6, p7, p8 = pch(5), pch(6), pch(7), pch(8)
        coor2 = ((p5 - lch(5)) ** 2 + (p6 - lch(6)) ** 2
                 + (jnp.sqrt(p7) - jnp.sqrt(lch(7))) ** 2
                 + (jnp.sqrt(p8) - jnp.sqrt(lch(8))) ** 2)
        iou2 = _iou_pair(p5, p6, p7, p8, l0, l1, l2, l3, a2)

        sel1 = iou1 >= iou2

        p4, p9, l4 = pch(4), pch(9), lch(4)
        obj = l4 == 1.0
        d1 = (p4 - iou1) ** 2
        d2 = (p9 - iou2) ** 2
        obj_branch = (5.0 * jnp.where(sel1, coor1, coor2)
                      + jnp.where(sel1, d1, d2)
                      + 0.5 * jnp.where(sel1, d2, d1)
                      + cls)
        noobj = 0.5 * (p4 * p4 + p9 * p9)
        per_cell = jnp.where(obj, obj_branch, noobj)

        out_ref[0:1, sl] += jnp.sum(per_cell, axis=0, keepdims=True)

    def chunk(k, carry):
        chunk_body(k)
        return carry

    jax.lax.fori_loop(0, _NCH, chunk, 0)


def kernel(pred, labels):
    pt = jnp.transpose(pred, (1, 2, 3, 0)).reshape(_CELLS, _C, _B)
    lt = jnp.transpose(labels, (1, 2, 3, 0)).reshape(_CELLS, _C, _B)

    in_spec = pl.BlockSpec((_S, _C, _BL), lambda j, i: (i, 0, j))

    out = pl.pallas_call(
        _yolo_kernel,
        out_shape=jax.ShapeDtypeStruct((1, _B), jnp.float32),
        grid=(_NL, _S),
        in_specs=[in_spec, in_spec],
        out_specs=pl.BlockSpec((1, _BL), lambda j, i: (0, j)),
        compiler_params=pltpu.CompilerParams(
            dimension_semantics=("arbitrary", "arbitrary"),
            vmem_limit_bytes=56 * 1024 * 1024,
        ),
        name="yolo_v1_loss",
    )(pt, lt)

    return jnp.sum(out) * (1.0 / _B)


# speedup vs baseline: 1.0516x; 1.0452x over previous
import jax
import jax.numpy as jnp
from jax.experimental import pallas as pl
from jax.experimental.pallas import tpu as pltpu

_B = 16384
_S = 7
_C = 30
_CELLS = _S * _S
_GX = 7.0
_GY = 30.0

_BL = 8192
_NL = _B // _BL
_CHUNK = 2048
_NCH = _BL // _CHUNK


def _iou_pair(pc0, pc1, pw, ph, lc0, lc1, lw, lh, a2):
    iw = jnp.minimum(jnp.minimum(
        0.5 * (pw + lw) - jnp.abs(pc0 - lc0) * (1.0 / _GX), pw), lw)
    ih = jnp.minimum(jnp.minimum(
        0.5 * (ph + lh) - jnp.abs(pc1 - lc1) * (1.0 / _GY), ph), lh)
    inter = jnp.maximum(iw, 0.0) * jnp.maximum(ih, 0.0)
    union = pw * ph + a2 - inter
    pos = inter > 0
    return jnp.where(pos, inter / jnp.where(pos, union, 1.0), 0.0)


def _yolo_kernel(p_ref, l_ref, out_ref, acc_ref):
    @pl.when((pl.program_id(0) == 0) & (pl.program_id(1) == 0))
    def _():
        acc_ref[...] = jnp.zeros_like(acc_ref)

    def chunk_body(k):
        sl = pl.ds(k * _CHUNK, _CHUNK)

        def pch(c):
            return p_ref[:, c, sl]

        def lch(c):
            return l_ref[:, c, sl]

        dd = p_ref[:, 10:30, sl] - l_ref[:, 10:30, sl]
        cls = jnp.sum(dd * dd, axis=1)

        l0, l1, l2, l3 = lch(0), lch(1), lch(2), lch(3)
        a2 = l2 * l3

        p0, p1, p2, p3 = pch(0), pch(1), pch(2), pch(3)
        coor1 = ((p0 - l0) ** 2 + (p1 - l1) ** 2
                 + (jnp.sqrt(p2) - jnp.sqrt(l2)) ** 2
                 + (jnp.sqrt(p3) - jnp.sqrt(l3)) ** 2)
        iou1 = _iou_pair(p0, p1, p2, p3, l0, l1, l2, l3, a2)

        p5, p6, p7, p8 = pch(5), pch(6), pch(7), pch(8)
        coor2 = ((p5 - lch(5)) ** 2 + (p6 - lch(6)) ** 2
                 + (jnp.sqrt(p7) - jnp.sqrt(lch(7))) ** 2
                 + (jnp.sqrt(p8) - jnp.sqrt(lch(8))) ** 2)
        iou2 = _iou_pair(p5, p6, p7, p8, l0, l1, l2, l3, a2)

        sel1 = iou1 >= iou2

        p4, p9, l4 = pch(4), pch(9), lch(4)
        obj = l4 == 1.0
        d1 = (p4 - iou1) ** 2
        d2 = (p9 - iou2) ** 2
        obj_branch = (5.0 * jnp.where(sel1, coor1, coor2)
                      + jnp.where(sel1, d1, d2)
                      + 0.5 * jnp.where(sel1, d2, d1)
                      + cls)
        noobj = 0.5 * (p4 * p4 + p9 * p9)
        per_cell = jnp.where(obj, obj_branch, noobj)

        acc_ref[0:_S, sl] += per_cell

    def chunk(k, carry):
        chunk_body(k)
        return carry

    jax.lax.fori_loop(0, _NCH, chunk, 0)

    @pl.when((pl.program_id(0) == _NL - 1) & (pl.program_id(1) == _S - 1))
    def _():
        total = jnp.sum(acc_ref[0:_S, :])
        out_ref[...] = jnp.zeros_like(out_ref) + total * (1.0 / _B)


def kernel(pred, labels):
    pt = jnp.transpose(pred, (1, 2, 3, 0)).reshape(_CELLS, _C, _B)
    lt = jnp.transpose(labels, (1, 2, 3, 0)).reshape(_CELLS, _C, _B)

    in_spec = pl.BlockSpec((_S, _C, _BL), lambda j, i: (i, 0, j))

    out = pl.pallas_call(
        _yolo_kernel,
        out_shape=jax.ShapeDtypeStruct((1, 128), jnp.float32),
        grid=(_NL, _S),
        in_specs=[in_spec, in_spec],
        out_specs=pl.BlockSpec((1, 128), lambda j, i: (0, 0)),
        scratch_shapes=[pltpu.VMEM((8, _BL), jnp.float32)],
        compiler_params=pltpu.CompilerParams(
            dimension_semantics=("arbitrary", "arbitrary"),
            vmem_limit_bytes=56 * 1024 * 1024,
        ),
        name="yolo_v1_loss",
    )(pt, lt)

    return out[0, 0]
